# Initial kernel scaffold; baseline (speedup 1.0000x reference)
import jax
import jax.numpy as jnp
from jax.experimental import pallas as pl


def kernel(wl_indices, x, edge_index, edge_weight, W_edge, b_edge, wl_table, node_W, node_b, cat1_W, cat1_b, cat2_W, cat2_b, final_W, final_b):
    raise NotImplementedError("write your pallas kernel here")


# speedup vs baseline: 4.4358x; 4.4358x over previous
import functools

import jax
import jax.numpy as jnp
from jax import lax
from jax.experimental import pallas as pl
from jax.experimental.pallas import tpu as pltpu
from jax.experimental.pallas import tpu_sc as plsc

N = 10000
E = 320000
H = 128
IN = 128
WL_DIM = 16
NUM_WL = 1000

NC = 2
NS = 16
NW = NC * NS

EPW = E // NW
CHUNK = 80
NCHUNK = EPW // CHUNK

ROWS_BIG = 640
ROWS_LAST = N - 15 * ROWS_BIG
ZROWS = 80

WL_WORKERS = 25
WL_PER_W = N // WL_WORKERS
WL_CHUNK = 80


def _sc_body(src_hbm, dst_hbm, ew_hbm, wedge_hbm, wlidx_hbm, wltab_hbm,
             part_hbm, wl_out_hbm,
             acc, zbuf, src_v, dst_v, w_v, rows_v, wl_idx_v, wl_rows_v, sem):
  c = lax.axis_index("c")
  s = lax.axis_index("s")
  w = c * NS + s

  zeros16 = jnp.zeros((16,), jnp.float32)

  def zero_row(r, _):
    for h in range(H // 16):
      zbuf[r, pl.ds(h * 16, 16)] = zeros16
    return 0

  lax.fori_loop(0, ZROWS, zero_row, 0)
  start = s * ROWS_BIG
  nz = jnp.where(s == NS - 1, ROWS_LAST // ZROWS, ROWS_BIG // ZROWS)

  def zero_chunk(j, _):
    pltpu.sync_copy(zbuf, acc.at[pl.ds(start + j * ZROWS, ZROWS)])
    return 0

  lax.fori_loop(0, nz, zero_chunk, 0)
  plsc.subcore_barrier()

  @pl.when(w < WL_WORKERS)
  def _():
    def wl_chunk(j, _):
      off = w * WL_PER_W + j * WL_CHUNK
      pltpu.sync_copy(wlidx_hbm.at[pl.ds(off, WL_CHUNK)], wl_idx_v)
      pltpu.async_copy(wltab_hbm.at[wl_idx_v], wl_rows_v, sem).wait()
      pltpu.sync_copy(wl_rows_v, wl_out_hbm.at[pl.ds(off, WL_CHUNK)])
      return 0

    lax.fori_loop(0, WL_PER_W // WL_CHUNK, wl_chunk, 0)

  base = w * EPW

  def edge_chunk(c0, _):
    off = base + c0 * CHUNK
    pltpu.sync_copy(src_hbm.at[pl.ds(off, CHUNK)], src_v)
    pltpu.sync_copy(dst_hbm.at[pl.ds(off, CHUNK)], dst_v)
    pltpu.sync_copy(ew_hbm.at[pl.ds(off, CHUNK)], w_v)
    pltpu.async_copy(wedge_hbm.at[src_v], rows_v, sem).wait()

    def scale_group(g, _):
      w16 = w_v[pl.ds(g * 16, 16)]
      for l in range(16):
        wspl = w16.at[jnp.full((16,), l, jnp.int32)].get(
            mode="promise_in_bounds")
        e = g * 16 + l
        for h in range(H // 16):
          rows_v[e, pl.ds(h * 16, 16)] = rows_v[e, pl.ds(h * 16, 16)] * wspl
      return 0

    lax.fori_loop(0, CHUNK // 16, scale_group, 0)
    pltpu.sync_copy(rows_v, acc.at[dst_v], add=True)
    return 0

  lax.fori_loop(0, NCHUNK, edge_chunk, 0)

  plsc.subcore_barrier()

  @pl.when(s < NS - 1)
  def _():
    pltpu.sync_copy(acc.at[pl.ds(s * ROWS_BIG, ROWS_BIG)],
                    part_hbm.at[c, pl.ds(s * ROWS_BIG, ROWS_BIG)])

  @pl.when(s == NS - 1)
  def _():
    pltpu.sync_copy(acc.at[pl.ds((NS - 1) * ROWS_BIG, ROWS_LAST)],
                    part_hbm.at[c, pl.ds((NS - 1) * ROWS_BIG, ROWS_LAST)])


def _sc_spmm(src, dst, ew, W_edge, wl_indices, wl_table):
  mesh = plsc.VectorSubcoreMesh(core_axis_name="c", subcore_axis_name="s",
                                num_cores=NC, num_subcores=NS)
  f = pl.kernel(
      _sc_body,
      out_type=(
          jax.ShapeDtypeStruct((NC, N, H), jnp.float32),
          jax.ShapeDtypeStruct((N, H), jnp.float32),
      ),
      mesh=mesh,
      scratch_types=[
          pltpu.VMEM_SHARED((N, H), jnp.float32),
          pltpu.VMEM((ZROWS, H), jnp.float32),
          pltpu.VMEM((CHUNK,), jnp.int32),
          pltpu.VMEM((CHUNK,), jnp.int32),
          pltpu.VMEM((CHUNK,), jnp.float32),
          pltpu.VMEM((CHUNK, H), jnp.float32),
          pltpu.VMEM((WL_CHUNK,), jnp.int32),
          pltpu.VMEM((WL_CHUNK, H), jnp.float32),
          pltpu.SemaphoreType.DMA,
      ],
  )
  return f(src, dst, ew, W_edge, wl_indices, wl_table)


BN = 2000


def _tc_body(part, x, wl_e, b_edge, c1W, c1b, nW0, nW1, nb, c2W, c2b, fW, fb,
             out):
  f32 = jnp.float32
  s1 = part[0] + part[1] + b_edge[:]
  t = s1 + jnp.dot(s1, c1W[:], preferred_element_type=f32) + c1b[:]
  xn = (jnp.dot(x[:], nW0[:], preferred_element_type=f32)
        + jnp.dot(wl_e[:, :WL_DIM], nW1[:], preferred_element_type=f32)
        + nb[:])
  t = t + xn + jnp.dot(xn, c2W[:], preferred_element_type=f32) + c2b[:]
  out[:] = jnp.dot(jnp.maximum(t, 0.0), fW[:], preferred_element_type=f32) + fb[:]


def _tc_dense(part, x, wl_e, b_edge, c1W, c1b, nW0, nW1, nb, c2W, c2b, fW, fb):
  grid = (N // BN,)
  in_specs = [
      pl.BlockSpec((NC, BN, H), lambda i: (0, i, 0)),
      pl.BlockSpec((BN, IN), lambda i: (i, 0)),
      pl.BlockSpec((BN, H), lambda i: (i, 0)),
      pl.BlockSpec((1, H), lambda i: (0, 0)),
      pl.BlockSpec((H, H), lambda i: (0, 0)),
      pl.BlockSpec((1, H), lambda i: (0, 0)),
      pl.BlockSpec((IN, H), lambda i: (0, 0)),
      pl.BlockSpec((WL_DIM, H), lambda i: (0, 0)),
      pl.BlockSpec((1, H), lambda i: (0, 0)),
      pl.BlockSpec((H, H), lambda i: (0, 0)),
      pl.BlockSpec((1, H), lambda i: (0, 0)),
      pl.BlockSpec((H, H), lambda i: (0, 0)),
      pl.BlockSpec((1, H), lambda i: (0, 0)),
  ]
  return pl.pallas_call(
      _tc_body,
      grid=grid,
      in_specs=in_specs,
      out_specs=pl.BlockSpec((BN, H), lambda i: (i, 0)),
      out_shape=jax.ShapeDtypeStruct((N, H), jnp.float32),
  )(part, x, wl_e, b_edge, c1W, c1b, nW0, nW1, nb, c2W, c2b, fW, fb)


def kernel(wl_indices, x, edge_index, edge_weight, W_edge, b_edge, wl_table,
           node_W, node_b, cat1_W, cat1_b, cat2_W, cat2_b, final_W, final_b):
  src = edge_index[0]
  dst = edge_index[1]
  wl_pad = jnp.pad(wl_table, ((0, 0), (0, H - WL_DIM)))
  part, wl_e = _sc_spmm(src, dst, edge_weight, W_edge, wl_indices, wl_pad)
  nW0 = node_W[:IN]
  nW1 = node_W[IN:]
  r = lambda v: v.reshape(1, -1)
  return _tc_dense(part, x, wl_e, r(b_edge), cat1_W, r(cat1_b), nW0, nW1,
                   r(node_b), cat2_W, r(cat2_b), final_W, r(final_b))


# speedup vs baseline: 4.7285x; 1.0660x over previous
import jax
import jax.numpy as jnp
from jax import lax
from jax.experimental import pallas as pl
from jax.experimental.pallas import tpu as pltpu
from jax.experimental.pallas import tpu_sc as plsc

N = 10000
E = 320000
H = 128
HH = H // 2
IN = 128
WL_DIM = 16
NUM_WL = 1000

NC = 2
NS = 16
NW = NC * NS

EPW = E // NW
CHUNK = 80
NCHUNK = EPW // CHUNK
RING = 5

ROWS_BIG = 640
ROWS_LAST = N - 15 * ROWS_BIG
ZROWS = 80

WL_WORKERS = 25
WL_PER_W = N // WL_WORKERS
WL_CHUNK = 80


def _sc_body(src_hbm, dst_hbm, ew_hbm, wA_hbm, wB_hbm, wlidx_hbm, wltab_hbm,
             partA_hbm, partB_hbm, wl_out_hbm,
             acc, src_s, dst_s, w_s,
             rows0, rows1, rows2, rows3, rows4, wl_idx_v,
             gsem0, gsem1, gsem2, gsem3, gsem4,
             ssem0, ssem1, ssem2, ssem3, ssem4, wlsem):
  c = lax.axis_index("c")
  s = lax.axis_index("s")
  w = c * NS + s
  rows = [rows0, rows1, rows2, rows3, rows4]
  gsems = [gsem0, gsem1, gsem2, gsem3, gsem4]
  ssems = [ssem0, ssem1, ssem2, ssem3, ssem4]

  zeros16 = jnp.zeros((16,), jnp.float32)

  pltpu.sync_copy(src_hbm.at[w], src_s)
  pltpu.sync_copy(dst_hbm.at[w], dst_s)
  pltpu.sync_copy(ew_hbm.at[w], w_s)

  @pl.when(w < WL_WORKERS)
  def _():
    def wl_chunk(j, _):
      off = w * WL_PER_W + j * WL_CHUNK
      pltpu.sync_copy(wlidx_hbm.at[pl.ds(off, WL_CHUNK)], wl_idx_v)
      pltpu.async_copy(wltab_hbm.at[wl_idx_v], rows0, wlsem).wait()
      pltpu.sync_copy(rows0, wl_out_hbm.at[pl.ds(off, WL_CHUNK)])
      return 0

    lax.fori_loop(0, WL_PER_W // WL_CHUNK, wl_chunk, 0)

  start = s * ROWS_BIG
  nz = jnp.where(s == NS - 1, ROWS_LAST // ZROWS, ROWS_BIG // ZROWS)

  def run_phase(wedge_hbm, part_hbm):
    def zero_row(r, _):
      for hh in range(HH // 16):
        rows0[r, pl.ds(hh * 16, 16)] = zeros16
      return 0

    lax.fori_loop(0, ZROWS, zero_row, 0)

    def zero_chunk(j, _):
      pltpu.sync_copy(rows0, acc.at[pl.ds(start + j * ZROWS, ZROWS)])
      return 0

    lax.fori_loop(0, nz, zero_chunk, 0)
    plsc.subcore_barrier()

    pltpu.async_copy(wedge_hbm.at[src_s.at[0]], rows[0], gsems[0])
    pltpu.async_copy(wedge_hbm.at[src_s.at[1]], rows[1], gsems[1])

    def quint(k, _):
      for b in range(RING):
        j = RING * k + b
        B = (b + 2) % RING
        pltpu.make_async_copy(wedge_hbm.at[pl.ds(0, CHUNK)], rows[b],
                              gsems[b]).wait()

        def scale_group(g, _, b=b, j=j):
          w16 = w_s[j, pl.ds(g * 16, 16)]
          for l in range(16):
            wspl = w16.at[jnp.full((16,), l, jnp.int32)].get(
                mode="promise_in_bounds")
            e = g * 16 + l
            for hh in range(HH // 16):
              rows[b][e, pl.ds(hh * 16, 16)] = (
                  rows[b][e, pl.ds(hh * 16, 16)] * wspl)
          return 0

        lax.fori_loop(0, CHUNK // 16, scale_group, 0)
        pltpu.async_copy(rows[b], acc.at[dst_s.at[j]], ssems[b], add=True)
        jj = j + 2

        @pl.when(jj < NCHUNK)
        def _(b=b, B=B, jj=jj):
          @pl.when(jj >= RING)
          def _():
            pltpu.make_async_copy(wedge_hbm.at[pl.ds(0, CHUNK)], rows[B],
                                  ssems[B]).wait()

          pltpu.async_copy(wedge_hbm.at[src_s.at[jj]], rows[B], gsems[B])

      return 0

    lax.fori_loop(0, NCHUNK // RING, quint, 0)
    for b in range(RING):
      pltpu.make_async_copy(wedge_hbm.at[pl.ds(0, CHUNK)], rows[b],
                            ssems[b]).wait()

    plsc.subcore_barrier()

    @pl.when(s < NS - 1)
    def _():
      pltpu.sync_copy(acc.at[pl.ds(s * ROWS_BIG, ROWS_BIG)],
                      part_hbm.at[c, pl.ds(s * ROWS_BIG, ROWS_BIG)])

    @pl.when(s == NS - 1)
    def _():
      pltpu.sync_copy(acc.at[pl.ds((NS - 1) * ROWS_BIG, ROWS_LAST)],
                      part_hbm.at[c, pl.ds((NS - 1) * ROWS_BIG, ROWS_LAST)])

  run_phase(wA_hbm, partA_hbm)
  run_phase(wB_hbm, partB_hbm)


def _sc_spmm(src3, dst3, ew3, W_A, W_B, wl_indices, wl_table_pad):
  mesh = plsc.VectorSubcoreMesh(core_axis_name="c", subcore_axis_name="s",
                                num_cores=NC, num_subcores=NS)
  f = pl.kernel(
      _sc_body,
      out_type=(
          jax.ShapeDtypeStruct((NC, N, HH), jnp.float32),
          jax.ShapeDtypeStruct((NC, N, HH), jnp.float32),
          jax.ShapeDtypeStruct((N, HH), jnp.float32),
      ),
      mesh=mesh,
      scratch_types=(
          [
              pltpu.VMEM_SHARED((N, HH), jnp.float32),
              pltpu.VMEM((NCHUNK, CHUNK), jnp.int32),
              pltpu.VMEM((NCHUNK, CHUNK), jnp.int32),
              pltpu.VMEM((NCHUNK, CHUNK), jnp.float32),
          ]
          + [pltpu.VMEM((CHUNK, HH), jnp.float32) for _ in range(RING)]
          + [pltpu.VMEM((WL_CHUNK,), jnp.int32)]
          + [pltpu.SemaphoreType.DMA for _ in range(2 * RING + 1)]
      ),
      compiler_params=pltpu.CompilerParams(use_tc_tiling_on_sc=False),
  )
  return f(src3, dst3, ew3, W_A, W_B, wl_indices, wl_table_pad)


BN = 2000


def _tc_body(pA, pB, x, wl_e, b_edge, c1W, c1b, nW0, nW1, nb, c2W, c2b,
             fW, fb, out):
  f32 = jnp.float32
  s1A = pA[0] + pA[1] + b_edge[0, :HH]
  s1B = pB[0] + pB[1] + b_edge[0, HH:]
  ri = lax.broadcasted_iota(jnp.int32, (HH, H), 0)
  ci = lax.broadcasted_iota(jnp.int32, (HH, H), 1)
  m1A = c1W[:HH, :] + (ci == ri).astype(f32)
  m1B = c1W[HH:, :] + (ci == ri + HH).astype(f32)
  t = (jnp.dot(s1A, m1A, preferred_element_type=f32)
       + jnp.dot(s1B, m1B, preferred_element_type=f32) + c1b[:])
  xn = (jnp.dot(x[:], nW0[:], preferred_element_type=f32)
        + jnp.dot(wl_e[:, :WL_DIM], nW1[:], preferred_element_type=f32)
        + nb[:])
  t = t + xn + jnp.dot(xn, c2W[:], preferred_element_type=f32) + c2b[:]
  out[:] = jnp.dot(jnp.maximum(t, 0.0), fW[:], preferred_element_type=f32) + fb[:]


def _tc_dense(pA, pB, x, wl_e, b_edge, c1W, c1b, nW0, nW1, nb, c2W, c2b,
              fW, fb):
  grid = (N // BN,)
  in_specs = [
      pl.BlockSpec((NC, BN, HH), lambda i: (0, i, 0)),
      pl.BlockSpec((NC, BN, HH), lambda i: (0, i, 0)),
      pl.BlockSpec((BN, IN), lambda i: (i, 0)),
      pl.BlockSpec((BN, HH), lambda i: (i, 0)),
      pl.BlockSpec((1, H), lambda i: (0, 0)),
      pl.BlockSpec((H, H), lambda i: (0, 0)),
      pl.BlockSpec((1, H), lambda i: (0, 0)),
      pl.BlockSpec((IN, H), lambda i: (0, 0)),
      pl.BlockSpec((WL_DIM, H), lambda i: (0, 0)),
      pl.BlockSpec((1, H), lambda i: (0, 0)),
      pl.BlockSpec((H, H), lambda i: (0, 0)),
      pl.BlockSpec((1, H), lambda i: (0, 0)),
      pl.BlockSpec((H, H), lambda i: (0, 0)),
      pl.BlockSpec((1, H), lambda i: (0, 0)),
  ]
  return pl.pallas_call(
      _tc_body,
      grid=grid,
      in_specs=in_specs,
      out_specs=pl.BlockSpec((BN, H), lambda i: (i, 0)),
      out_shape=jax.ShapeDtypeStruct((N, H), jnp.float32),
  )(pA, pB, x, wl_e, b_edge, c1W, c1b, nW0, nW1, nb, c2W, c2b, fW, fb)


def kernel(wl_indices, x, edge_index, edge_weight, W_edge, b_edge, wl_table,
           node_W, node_b, cat1_W, cat1_b, cat2_W, cat2_b, final_W, final_b):
  src3 = edge_index[0].reshape(NW, NCHUNK, CHUNK)
  dst3 = edge_index[1].reshape(NW, NCHUNK, CHUNK)
  ew3 = edge_weight.reshape(NW, NCHUNK, CHUNK)
  W_A = W_edge[:, :HH]
  W_B = W_edge[:, HH:]
  wl_pad = jnp.pad(wl_table, ((0, 0), (0, HH - WL_DIM)))
  pA, pB, wl_e = _sc_spmm(src3, dst3, ew3, W_A, W_B, wl_indices, wl_pad)
  nW0 = node_W[:IN]
  nW1 = node_W[IN:]
  r = lambda v: v.reshape(1, -1)
  return _tc_dense(pA, pB, x, wl_e, r(b_edge), cat1_W, r(cat1_b), nW0, nW1,
                   r(node_b), cat2_W, r(cat2_b), final_W, r(final_b))
